# baseline (speedup 1.0000x reference)
import jax
import jax.numpy as jnp
from jax.experimental import pallas as pl
from jax.experimental.pallas import tpu as pltpu

EPS = 1e-05

C = 128
HW = 4096
SLAB = C * HW


def _block_stats(ref):
    ssum = None
    ssumsq = None
    for c in range(C):
        v = ref[pl.ds(HW * c, HW)]
        q = v * v
        if c == 0:
            ssum, ssumsq = v, q
        else:
            ssum = ssum + v
            ssumsq = ssumsq + q
    return ssum, ssumsq


def _spe_kernel(s_ref, xs_ref, xn_ref, out_ref):
    sum_n, sumsq_n = _block_stats(xn_ref)
    sum_s, sumsq_s = _block_stats(xs_ref)

    mean_n = sum_n * (1.0 / C)
    var_n = (sumsq_n - sum_n * mean_n) * (1.0 / (C - 1))
    mean_s = sum_s * (1.0 / C)
    var_s = (sumsq_s - sum_s * mean_s) * (1.0 / (C - 1))

    ratio = jnp.sqrt((var_n + EPS) / (var_s + EPS))
    offset = mean_n - mean_s * ratio

    for c in range(C):
        w = pl.ds(HW * c, HW)
        out_ref[w] = xs_ref[w] * ratio + offset


def kernel(x, idx_swap):
    N = x.shape[0]
    x1 = x.reshape(N * SLAB)

    grid_spec = pltpu.PrefetchScalarGridSpec(
        num_scalar_prefetch=1,
        grid=(N,),
        in_specs=[
            pl.BlockSpec((SLAB,), lambda n, s: (s[n],)),
            pl.BlockSpec((SLAB,), lambda n, s: (n,)),
        ],
        out_specs=pl.BlockSpec((SLAB,), lambda n, s: (n,)),
    )

    out = pl.pallas_call(
        _spe_kernel,
        grid_spec=grid_spec,
        out_shape=jax.ShapeDtypeStruct((N * SLAB,), jnp.float32),
    )(idx_swap, x1, x1)
    return out.reshape(x.shape)
